# Initial kernel scaffold; baseline (speedup 1.0000x reference)
import jax
import jax.numpy as jnp
from jax.experimental import pallas as pl


def kernel(I1, I2, I3, I4, I5, I6, I7, I8, I9, I10, I11, I12, I13, ln_gamma_I1, ln_gamma_I2, ln_gamma_I3, ln_gamma_I4, ln_gamma_I5, ln_gamma_I6, ln_gamma_I7, ln_gamma_I8, ln_gamma_I9, ln_gamma_I10, ln_gamma_I11, ln_gamma_I12, ln_gamma_I13, ln_beta_I1, ln_beta_I2, ln_beta_I3, ln_beta_I4, ln_beta_I5, ln_beta_I6, ln_beta_I7, ln_beta_I8, ln_beta_I9, ln_beta_I10, ln_beta_I11, ln_beta_I12, ln_beta_I13, C1, C2, C3, C4, C5, C6, C7, C8, C9, C10, C11, C12, C13, C14, C15, C16, C17, C18, C19, C20, C21, C22, C23, C24, C25, C26, emb_C1, emb_C2, emb_C3, emb_C4, emb_C5, emb_C6, emb_C7, emb_C8, emb_C9, emb_C10, emb_C11, emb_C12, emb_C13, emb_C14, emb_C15, emb_C16, emb_C17, emb_C18, emb_C19, emb_C20, emb_C21, emb_C22, emb_C23, emb_C24, emb_C25, emb_C26, W, b):
    raise NotImplementedError("write your pallas kernel here")


# speedup vs baseline: 2.2964x; 2.2964x over previous
import functools

import jax
import jax.numpy as jnp
from jax import lax
from jax.experimental import pallas as pl
from jax.experimental.pallas import tpu as pltpu
from jax.experimental.pallas import tpu_sc as plsc

BATCH = 16384
DIM = 16
NTAB = 26
NC = 2
NS = 16
L = 16
NW = NC * NS
BPW = BATCH // NW


def _sc_body(*refs):
    idx_hbm = refs[0]
    bw_hbm = refs[1]
    w_hbm = refs[2]
    tables = refs[3:3 + NTAB]
    out_hbm = refs[3 + NTAB]
    (idx_v, rows_v, vacc, w_v, bw_v, out_v, sem) = refs[4 + NTAB:]

    wid = lax.axis_index("s") * NC + lax.axis_index("c")
    base = wid * BPW

    pltpu.sync_copy(w_hbm, w_v)
    pltpu.sync_copy(bw_hbm, bw_v)
    prod = bw_v[0] * bw_v[1]
    const = jnp.float32(0.0)
    for i in range(L):
        const = const + prod[i]

    for j in range(NTAB):
        pltpu.sync_copy(idx_hbm.at[j, pl.ds(base, BPW)], idx_v)
        pltpu.async_copy(tables[j].at[idx_v], rows_v, sem).wait()
        wj = w_v[j]

        @pl.loop(0, BPW, unroll=8)
        def _(b, j=j, wj=wj):
            off = pl.multiple_of(b * DIM, DIM)
            if j == 0:
                vacc[pl.ds(off, DIM)] = rows_v[b] * wj
            else:
                vacc[pl.ds(off, DIM)] = vacc[pl.ds(off, DIM)] + rows_v[b] * wj

    iota16 = lax.iota(jnp.int32, L) * DIM

    @pl.loop(0, BPW // L)
    def _(g):
        acc = jnp.zeros((L,), jnp.float32)
        for d in range(DIM):
            acc = acc + plsc.load_gather(vacc, [iota16 + (g * L * DIM + d)])
        x = acc + const
        y = 1.0 / (1.0 + jnp.exp(-x))
        out_v[pl.ds(pl.multiple_of(g * L, L), L)] = y

    pltpu.sync_copy(out_v, out_hbm.at[pl.ds(base, BPW)])


_lr_sc = functools.partial(
    pl.kernel,
    out_type=jax.ShapeDtypeStruct((BATCH,), jnp.float32),
    mesh=plsc.VectorSubcoreMesh(
        core_axis_name="c", subcore_axis_name="s", num_cores=NC, num_subcores=NS
    ),
    compiler_params=pltpu.CompilerParams(
        needs_layout_passes=False, use_tc_tiling_on_sc=False
    ),
    scratch_types=[
        pltpu.VMEM((BPW,), jnp.int32),
        pltpu.VMEM((BPW, DIM), jnp.float32),
        pltpu.VMEM((BPW * DIM,), jnp.float32),
        pltpu.VMEM((NTAB, DIM), jnp.float32),
        pltpu.VMEM((2, L), jnp.float32),
        pltpu.VMEM((BPW,), jnp.float32),
        pltpu.SemaphoreType.DMA,
    ],
)(_sc_body)


def kernel(I1, I2, I3, I4, I5, I6, I7, I8, I9, I10, I11, I12, I13, ln_gamma_I1, ln_gamma_I2, ln_gamma_I3, ln_gamma_I4, ln_gamma_I5, ln_gamma_I6, ln_gamma_I7, ln_gamma_I8, ln_gamma_I9, ln_gamma_I10, ln_gamma_I11, ln_gamma_I12, ln_gamma_I13, ln_beta_I1, ln_beta_I2, ln_beta_I3, ln_beta_I4, ln_beta_I5, ln_beta_I6, ln_beta_I7, ln_beta_I8, ln_beta_I9, ln_beta_I10, ln_beta_I11, ln_beta_I12, ln_beta_I13, C1, C2, C3, C4, C5, C6, C7, C8, C9, C10, C11, C12, C13, C14, C15, C16, C17, C18, C19, C20, C21, C22, C23, C24, C25, C26, emb_C1, emb_C2, emb_C3, emb_C4, emb_C5, emb_C6, emb_C7, emb_C8, emb_C9, emb_C10, emb_C11, emb_C12, emb_C13, emb_C14, emb_C15, emb_C16, emb_C17, emb_C18, emb_C19, emb_C20, emb_C21, emb_C22, emb_C23, emb_C24, emb_C25, emb_C26, W, b):
    Cs = [C1, C2, C3, C4, C5, C6, C7, C8, C9, C10, C11, C12, C13, C14,
          C15, C16, C17, C18, C19, C20, C21, C22, C23, C24, C25, C26]
    tabs = [emb_C1, emb_C2, emb_C3, emb_C4, emb_C5, emb_C6, emb_C7, emb_C8,
            emb_C9, emb_C10, emb_C11, emb_C12, emb_C13, emb_C14, emb_C15,
            emb_C16, emb_C17, emb_C18, emb_C19, emb_C20, emb_C21, emb_C22,
            emb_C23, emb_C24, emb_C25, emb_C26]
    betas = [ln_beta_I1, ln_beta_I2, ln_beta_I3, ln_beta_I4, ln_beta_I5,
             ln_beta_I6, ln_beta_I7, ln_beta_I8, ln_beta_I9, ln_beta_I10,
             ln_beta_I11, ln_beta_I12, ln_beta_I13]

    idx_all = jnp.concatenate([c.reshape(1, BATCH) for c in Cs], axis=0)
    beta_pad = jnp.concatenate(betas + [b, jnp.zeros((2,), jnp.float32)])
    wd_pad = jnp.concatenate([W[:13, 0], jnp.ones((1,), jnp.float32),
                              jnp.zeros((2,), jnp.float32)])
    bw = jnp.stack([beta_pad, wd_pad])
    w_sp = W[13:, 0].reshape(NTAB, DIM)

    out = _lr_sc(idx_all, bw, w_sp, *tabs)
    return out.reshape(BATCH, 1)


# speedup vs baseline: 17.1160x; 7.4535x over previous
import functools

import jax
import jax.numpy as jnp
from jax import lax
from jax.experimental import pallas as pl
from jax.experimental.pallas import tpu as pltpu
from jax.experimental.pallas import tpu_sc as plsc

BATCH = 16384
VOCAB = 100000
DIM = 16
NTAB = 26
NC = 2
NS = 16
L = 16
NW = NC * NS
BPW = BATCH // NW

BLKV = 4096
NBLK = 25
VPAD = NBLK * BLKV


def _scores_body(*refs):
    w_ref = refs[0]
    tab_refs = refs[1:1 + NTAB]
    out_refs = refs[1 + NTAB:]
    for j in range(NTAB):
        wj = w_ref[j, :].reshape(1, DIM)
        s = jnp.dot(wj, tab_refs[j][...], preferred_element_type=jnp.float32)
        out_refs[j][...] = s.reshape(BLKV)


def _make_scores_call():
    in_specs = [pl.BlockSpec((NTAB, DIM), lambda i: (0, 0))]
    in_specs += [pl.BlockSpec((DIM, BLKV), lambda i: (0, i))
                 for _ in range(NTAB)]
    out_specs = [pl.BlockSpec((BLKV,), lambda i: (i,)) for _ in range(NTAB)]
    return pl.pallas_call(
        _scores_body,
        grid=(NBLK,),
        in_specs=in_specs,
        out_specs=out_specs,
        out_shape=[jax.ShapeDtypeStruct((VPAD,), jnp.float32)
                   for _ in range(NTAB)],
    )


_scores_tc = _make_scores_call()


def _gather_body(*refs):
    idx_hbm = refs[0]
    w_hbm = refs[1]
    score_refs = refs[2:2 + NTAB]
    out_hbm = refs[2 + NTAB]
    (idx_v, vals_v, acc_v, w_v, out_v, sem) = refs[3 + NTAB:]

    wid = lax.axis_index("s") * NC + lax.axis_index("c")
    base = wid * BPW

    pltpu.sync_copy(w_hbm, w_v)
    prod = w_v[pl.ds(0, L)] * w_v[pl.ds(L, L)]
    const = jnp.float32(0.0)
    for i in range(L):
        const = const + prod[i]

    for j in range(NTAB):
        pltpu.sync_copy(idx_hbm.at[pl.ds(j * BATCH + base, BPW)], idx_v)
        pltpu.async_copy(score_refs[j].at[idx_v], vals_v, sem).wait()

        @pl.loop(0, BPW // L, unroll=8)
        def _(k, j=j):
            off = pl.multiple_of(k * L, L)
            if j == 0:
                acc_v[pl.ds(off, L)] = vals_v[pl.ds(off, L)]
            else:
                acc_v[pl.ds(off, L)] = (acc_v[pl.ds(off, L)]
                                        + vals_v[pl.ds(off, L)])

    @pl.loop(0, BPW // L, unroll=4)
    def _(k):
        off = pl.multiple_of(k * L, L)
        x = acc_v[pl.ds(off, L)] + const
        out_v[pl.ds(off, L)] = 1.0 / (1.0 + jnp.exp(-x))

    pltpu.sync_copy(out_v, out_hbm.at[pl.ds(base, BPW)])


_gather_sc = functools.partial(
    pl.kernel,
    out_type=jax.ShapeDtypeStruct((BATCH,), jnp.float32),
    mesh=plsc.VectorSubcoreMesh(
        core_axis_name="c", subcore_axis_name="s", num_cores=NC, num_subcores=NS
    ),
    compiler_params=pltpu.CompilerParams(
        needs_layout_passes=False, use_tc_tiling_on_sc=False
    ),
    scratch_types=[
        pltpu.VMEM((BPW,), jnp.int32),
        pltpu.VMEM((BPW,), jnp.float32),
        pltpu.VMEM((BPW,), jnp.float32),
        pltpu.VMEM((2 * L,), jnp.float32),
        pltpu.VMEM((BPW,), jnp.float32),
        pltpu.SemaphoreType.DMA,
    ],
)(_gather_body)


def kernel(I1, I2, I3, I4, I5, I6, I7, I8, I9, I10, I11, I12, I13, ln_gamma_I1, ln_gamma_I2, ln_gamma_I3, ln_gamma_I4, ln_gamma_I5, ln_gamma_I6, ln_gamma_I7, ln_gamma_I8, ln_gamma_I9, ln_gamma_I10, ln_gamma_I11, ln_gamma_I12, ln_gamma_I13, ln_beta_I1, ln_beta_I2, ln_beta_I3, ln_beta_I4, ln_beta_I5, ln_beta_I6, ln_beta_I7, ln_beta_I8, ln_beta_I9, ln_beta_I10, ln_beta_I11, ln_beta_I12, ln_beta_I13, C1, C2, C3, C4, C5, C6, C7, C8, C9, C10, C11, C12, C13, C14, C15, C16, C17, C18, C19, C20, C21, C22, C23, C24, C25, C26, emb_C1, emb_C2, emb_C3, emb_C4, emb_C5, emb_C6, emb_C7, emb_C8, emb_C9, emb_C10, emb_C11, emb_C12, emb_C13, emb_C14, emb_C15, emb_C16, emb_C17, emb_C18, emb_C19, emb_C20, emb_C21, emb_C22, emb_C23, emb_C24, emb_C25, emb_C26, W, b):
    Cs = [C1, C2, C3, C4, C5, C6, C7, C8, C9, C10, C11, C12, C13, C14,
          C15, C16, C17, C18, C19, C20, C21, C22, C23, C24, C25, C26]
    tabs = [emb_C1, emb_C2, emb_C3, emb_C4, emb_C5, emb_C6, emb_C7, emb_C8,
            emb_C9, emb_C10, emb_C11, emb_C12, emb_C13, emb_C14, emb_C15,
            emb_C16, emb_C17, emb_C18, emb_C19, emb_C20, emb_C21, emb_C22,
            emb_C23, emb_C24, emb_C25, emb_C26]
    betas = [ln_beta_I1, ln_beta_I2, ln_beta_I3, ln_beta_I4, ln_beta_I5,
             ln_beta_I6, ln_beta_I7, ln_beta_I8, ln_beta_I9, ln_beta_I10,
             ln_beta_I11, ln_beta_I12, ln_beta_I13]

    idx_all = jnp.concatenate([c.reshape(BATCH) for c in Cs])
    beta_pad = jnp.concatenate(betas + [b, jnp.zeros((2,), jnp.float32)])
    wd_pad = jnp.concatenate([W[:13, 0], jnp.ones((1,), jnp.float32),
                              jnp.zeros((2,), jnp.float32)])
    w_head = jnp.concatenate([beta_pad, wd_pad])
    w_sp = W[13:, 0].reshape(NTAB, DIM)

    scores = _scores_tc(w_sp, *(t.T for t in tabs))
    out = _gather_sc(idx_all, w_head, *scores)
    return out.reshape(BATCH, 1)


# speedup vs baseline: 24.8620x; 1.4526x over previous
import functools

import jax
import jax.numpy as jnp
from jax import lax
from jax.experimental import pallas as pl
from jax.experimental.pallas import tpu as pltpu
from jax.experimental.pallas import tpu_sc as plsc

BATCH = 16384
VOCAB = 100000
DIM = 16
NTAB = 26
NC = 2
NS = 16
L = 16
NW = NC * NS
BPW = BATCH // NW

BLKV = 4096
NBLK = 25
VPAD = NBLK * BLKV


def _scores_body(*refs):
    w_ref = refs[0]
    tab_refs = refs[1:1 + NTAB]
    out_refs = refs[1 + NTAB:]
    for j in range(NTAB):
        wj = w_ref[j, :].reshape(1, DIM)
        s = jnp.dot(wj, tab_refs[j][...], preferred_element_type=jnp.float32)
        out_refs[j][...] = s.reshape(BLKV)


def _make_scores_call():
    in_specs = [pl.BlockSpec((NTAB, DIM), lambda i: (0, 0))]
    in_specs += [pl.BlockSpec((DIM, BLKV), lambda i: (0, i))
                 for _ in range(NTAB)]
    out_specs = [pl.BlockSpec((BLKV,), lambda i: (i,)) for _ in range(NTAB)]
    return pl.pallas_call(
        _scores_body,
        grid=(NBLK,),
        in_specs=in_specs,
        out_specs=out_specs,
        out_shape=[jax.ShapeDtypeStruct((VPAD,), jnp.float32)
                   for _ in range(NTAB)],
    )


_scores_tc = _make_scores_call()


def _gather_body(*refs):
    idx_refs = refs[0:NTAB]
    w_hbm = refs[NTAB]
    score_refs = refs[NTAB + 1:2 * NTAB + 1]
    out_hbm = refs[2 * NTAB + 1]
    (idx_v, vals_v, w_v, out_v, sem, sem2) = refs[2 * NTAB + 2:]

    wid = lax.axis_index("s") * NC + lax.axis_index("c")
    base = wid * BPW

    pltpu.sync_copy(w_hbm, w_v)
    prod = w_v[pl.ds(0, L)] * w_v[pl.ds(L, L)]
    const = jnp.float32(0.0)
    for i in range(L):
        const = const + prod[i]

    idx_copies = [
        pltpu.async_copy(idx_refs[j].at[pl.ds(base, BPW)], idx_v.at[j], sem)
        for j in range(NTAB)
    ]
    for cp in idx_copies:
        cp.wait()
    gathers = [
        pltpu.async_copy(score_refs[j].at[idx_v.at[j]], vals_v.at[j], sem2)
        for j in range(NTAB)
    ]
    for cp in gathers:
        cp.wait()

    @pl.loop(0, BPW // L)
    def _(k):
        off = pl.multiple_of(k * L, L)
        acc = vals_v[0, pl.ds(off, L)] + const
        for j in range(1, NTAB):
            acc = acc + vals_v[j, pl.ds(off, L)]
        out_v[pl.ds(off, L)] = 1.0 / (1.0 + jnp.exp(-acc))

    pltpu.sync_copy(out_v, out_hbm.at[pl.ds(base, BPW)])


_gather_sc = functools.partial(
    pl.kernel,
    out_type=jax.ShapeDtypeStruct((BATCH,), jnp.float32),
    mesh=plsc.VectorSubcoreMesh(
        core_axis_name="c", subcore_axis_name="s", num_cores=NC, num_subcores=NS
    ),
    compiler_params=pltpu.CompilerParams(
        needs_layout_passes=False, use_tc_tiling_on_sc=False
    ),
    scratch_types=[
        pltpu.VMEM((NTAB, BPW), jnp.int32),
        pltpu.VMEM((NTAB, BPW), jnp.float32),
        pltpu.VMEM((2 * L,), jnp.float32),
        pltpu.VMEM((BPW,), jnp.float32),
        pltpu.SemaphoreType.DMA,
        pltpu.SemaphoreType.DMA,
    ],
)(_gather_body)


def kernel(I1, I2, I3, I4, I5, I6, I7, I8, I9, I10, I11, I12, I13, ln_gamma_I1, ln_gamma_I2, ln_gamma_I3, ln_gamma_I4, ln_gamma_I5, ln_gamma_I6, ln_gamma_I7, ln_gamma_I8, ln_gamma_I9, ln_gamma_I10, ln_gamma_I11, ln_gamma_I12, ln_gamma_I13, ln_beta_I1, ln_beta_I2, ln_beta_I3, ln_beta_I4, ln_beta_I5, ln_beta_I6, ln_beta_I7, ln_beta_I8, ln_beta_I9, ln_beta_I10, ln_beta_I11, ln_beta_I12, ln_beta_I13, C1, C2, C3, C4, C5, C6, C7, C8, C9, C10, C11, C12, C13, C14, C15, C16, C17, C18, C19, C20, C21, C22, C23, C24, C25, C26, emb_C1, emb_C2, emb_C3, emb_C4, emb_C5, emb_C6, emb_C7, emb_C8, emb_C9, emb_C10, emb_C11, emb_C12, emb_C13, emb_C14, emb_C15, emb_C16, emb_C17, emb_C18, emb_C19, emb_C20, emb_C21, emb_C22, emb_C23, emb_C24, emb_C25, emb_C26, W, b):
    Cs = [C1, C2, C3, C4, C5, C6, C7, C8, C9, C10, C11, C12, C13, C14,
          C15, C16, C17, C18, C19, C20, C21, C22, C23, C24, C25, C26]
    tabs = [emb_C1, emb_C2, emb_C3, emb_C4, emb_C5, emb_C6, emb_C7, emb_C8,
            emb_C9, emb_C10, emb_C11, emb_C12, emb_C13, emb_C14, emb_C15,
            emb_C16, emb_C17, emb_C18, emb_C19, emb_C20, emb_C21, emb_C22,
            emb_C23, emb_C24, emb_C25, emb_C26]
    betas = [ln_beta_I1, ln_beta_I2, ln_beta_I3, ln_beta_I4, ln_beta_I5,
             ln_beta_I6, ln_beta_I7, ln_beta_I8, ln_beta_I9, ln_beta_I10,
             ln_beta_I11, ln_beta_I12, ln_beta_I13]

    beta_pad = jnp.concatenate(betas + [b, jnp.zeros((2,), jnp.float32)])
    wd_pad = jnp.concatenate([W[:13, 0], jnp.ones((1,), jnp.float32),
                              jnp.zeros((2,), jnp.float32)])
    w_head = jnp.concatenate([beta_pad, wd_pad])
    w_sp = W[13:, 0].reshape(NTAB, DIM)

    scores = _scores_tc(w_sp, *(t.T for t in tabs))
    out = _gather_sc(*(c.reshape(BATCH) for c in Cs), w_head, *scores)
    return out.reshape(BATCH, 1)
